# baseline (speedup 1.0000x reference)
import functools

import jax
import jax.numpy as jnp
from jax import lax
from jax.experimental import pallas as pl
from jax.experimental.pallas import tpu as pltpu
from jax.experimental.pallas import tpu_sc as plsc

_LANES = 16
_NBUF = 8
_LAG = 4


def _booleanize_sc(mask_flat):
    n = mask_flat.shape[0]
    info = plsc.get_sparse_core_info()
    nc, ns = info.num_cores, info.num_subcores
    nw = nc * ns
    per_w = n // nw
    assert per_w % _LANES == 0 and n % nw == 0

    mesh = plsc.VectorSubcoreMesh(core_axis_name="c", subcore_axis_name="s")

    @functools.partial(
        pl.kernel,
        mesh=mesh,
        out_type=jax.ShapeDtypeStruct((n,), jnp.int32),
        compiler_params=pltpu.CompilerParams(needs_layout_passes=False),
        scratch_types=[
            pltpu.VMEM((per_w,), jnp.float32),
            pltpu.VMEM((per_w,), jnp.int32),
        ],
    )
    def k(m_hbm, out_hbm, m_v, o_v):
        wid = lax.axis_index("s") * nc + lax.axis_index("c")
        base = wid * per_w
        pltpu.sync_copy(m_hbm.at[pl.ds(base, per_w)], m_v)
        for i in range(per_w // _LANES):
            sl = pl.ds(i * _LANES, _LANES)
            o_v[sl] = (m_v[sl] != 0.0).astype(jnp.int32)
        pltpu.sync_copy(o_v, out_hbm.at[pl.ds(base, per_w)])

    return k(mask_flat)


def _copy_u_tc(U):
    R = 8
    Uf = U.reshape(-1, U.shape[-2], U.shape[-1])
    n = Uf.shape[0] // R

    def body(u_in, u_out):
        u_out[...] = u_in[...]

    out = pl.pallas_call(
        body,
        grid=(n,),
        in_specs=[pl.BlockSpec((R, 512, 512), lambda i: (i, 0, 0))],
        out_specs=pl.BlockSpec((R, 512, 512), lambda i: (i, 0, 0)),
        out_shape=jax.ShapeDtypeStruct(Uf.shape, Uf.dtype),
        compiler_params=pltpu.CompilerParams(
            dimension_semantics=("parallel",),
            vmem_limit_bytes=100 * 1024 * 1024,
        ),
    )(Uf)
    return out.reshape(U.shape)


def _copy_xv_tc(x, v):
    B = x.shape[0]

    def body(x_in, v_in, x_out, v_out):
        x_out[...] = x_in[...]
        v_out[...] = v_in[...]

    return pl.pallas_call(
        body,
        grid=(B,),
        in_specs=[
            pl.BlockSpec((1,) + x.shape[1:], lambda i: (i, 0, 0)),
            pl.BlockSpec((1,) + v.shape[1:], lambda i: (i, 0, 0)),
        ],
        out_specs=[
            pl.BlockSpec((1,) + x.shape[1:], lambda i: (i, 0, 0)),
            pl.BlockSpec((1,) + v.shape[1:], lambda i: (i, 0, 0)),
        ],
        out_shape=[
            jax.ShapeDtypeStruct(x.shape, x.dtype),
            jax.ShapeDtypeStruct(v.shape, v.dtype),
        ],
        compiler_params=pltpu.CompilerParams(
            dimension_semantics=("arbitrary",),
        ),
    )(x, v)


def kernel(x, v, mask, U):
    mi = _booleanize_sc(mask.reshape(-1))
    ox, ov = _copy_xv_tc(x, v)
    U_gated = lax.optimization_barrier((U, ox, ov))[0]
    oU = _copy_u_tc(U_gated)
    mb = mi.astype(jnp.bool_).reshape(mask.shape)
    return (ox, ov, mb, oU)


# speedup vs baseline: 1.0794x; 1.0794x over previous
import functools

import jax
import jax.numpy as jnp
from jax import lax
from jax.experimental import pallas as pl
from jax.experimental.pallas import tpu as pltpu
from jax.experimental.pallas import tpu_sc as plsc

_LANES = 16
_NBUF = 8
_LAG = 4


def _booleanize_sc(mask_flat):
    n = mask_flat.shape[0]
    info = plsc.get_sparse_core_info()
    nc, ns = info.num_cores, info.num_subcores
    nw = nc * ns
    per_w = n // nw
    assert per_w % _LANES == 0 and n % nw == 0

    mesh = plsc.VectorSubcoreMesh(core_axis_name="c", subcore_axis_name="s")

    @functools.partial(
        pl.kernel,
        mesh=mesh,
        out_type=jax.ShapeDtypeStruct((n,), jnp.int32),
        compiler_params=pltpu.CompilerParams(needs_layout_passes=False),
        scratch_types=[
            pltpu.VMEM((per_w,), jnp.float32),
            pltpu.VMEM((per_w,), jnp.int32),
        ],
    )
    def k(m_hbm, out_hbm, m_v, o_v):
        wid = lax.axis_index("s") * nc + lax.axis_index("c")
        base = wid * per_w
        pltpu.sync_copy(m_hbm.at[pl.ds(base, per_w)], m_v)
        for i in range(per_w // _LANES):
            sl = pl.ds(i * _LANES, _LANES)
            o_v[sl] = (m_v[sl] != 0.0).astype(jnp.int32)
        pltpu.sync_copy(o_v, out_hbm.at[pl.ds(base, per_w)])

    return k(mask_flat)


def _passthrough_tc(x, v, U):
    R = 8
    Uf = U.reshape(-1, U.shape[-2], U.shape[-1])
    n = Uf.shape[0] // R

    def body(u_in, x_in, v_in, u_out, x_out, v_out):
        u_out[...] = u_in[...]
        x_out[...] = x_in[...]
        v_out[...] = v_in[...]

    ou, ox, ov = pl.pallas_call(
        body,
        grid=(n,),
        in_specs=[
            pl.BlockSpec((R, 512, 512), lambda i: (i, 0, 0)),
            pl.BlockSpec((1,) + x.shape[1:], lambda i: (i, 0, 0)),
            pl.BlockSpec((1,) + v.shape[1:], lambda i: (i, 0, 0)),
        ],
        out_specs=[
            pl.BlockSpec((R, 512, 512), lambda i: (i, 0, 0)),
            pl.BlockSpec((1,) + x.shape[1:], lambda i: (i, 0, 0)),
            pl.BlockSpec((1,) + v.shape[1:], lambda i: (i, 0, 0)),
        ],
        out_shape=[
            jax.ShapeDtypeStruct(Uf.shape, Uf.dtype),
            jax.ShapeDtypeStruct(x.shape, x.dtype),
            jax.ShapeDtypeStruct(v.shape, v.dtype),
        ],
        compiler_params=pltpu.CompilerParams(
            dimension_semantics=("arbitrary",),
            vmem_limit_bytes=100 * 1024 * 1024,
        ),
    )(Uf, x, v)
    return ou.reshape(U.shape), ox, ov


def kernel(x, v, mask, U):
    mi = _booleanize_sc(mask.reshape(-1))
    oU, ox, ov = _passthrough_tc(x, v, U)
    mb = mi.astype(jnp.bool_).reshape(mask.shape)
    return (ox, ov, mb, oU)
